# baseline (speedup 1.0000x reference)
import jax
import jax.numpy as jnp
from jax import lax
from jax.experimental import pallas as pl
from jax.experimental.pallas import tpu as pltpu
from jax.experimental.pallas import tpu_sc as plsc

BATCH = 16384
EMB = 32
SCOLS = EMB + 1
NC = 2
NS = 16
NW = NC * NS
B_PER_W = BATCH // NW
GROUPS = B_PER_W // 16


def _gather_body(uid_hbm, iid_hbm, uemb_hbm, iemb_hbm, ubias_hbm, ibias_hbm,
                 stagu_hbm, stagi_hbm, stagub_hbm, stagib_hbm,
                 uid_v, iid_v, sem):
    wid = lax.axis_index("s") * NC + lax.axis_index("c")
    base = wid * B_PER_W

    pltpu.sync_copy(uid_hbm.at[pl.ds(wid * 4, 4)], uid_v)
    pltpu.sync_copy(iid_hbm.at[pl.ds(wid * 4, 4)], iid_v)

    def issue(g, carry):
        uids = uid_v[g // 8, pl.ds((g % 8) * 16, 16)]
        iids = iid_v[g // 8, pl.ds((g % 8) * 16, 16)]
        for i in range(16):
            ur = uids[i]
            ir = iids[i]
            e = base + g * 16 + i
            pltpu.async_copy(uemb_hbm.at[pl.ds(ur, 1), :],
                             stagu_hbm.at[pl.ds(e, 1), :], sem)
            pltpu.async_copy(iemb_hbm.at[pl.ds(ir, 1), :],
                             stagi_hbm.at[pl.ds(e, 1), :], sem)
            pltpu.async_copy(ubias_hbm.at[pl.ds(ur, 1), :],
                             stagub_hbm.at[pl.ds(e, 1), :], sem)
            pltpu.async_copy(ibias_hbm.at[pl.ds(ir, 1), :],
                             stagib_hbm.at[pl.ds(e, 1), :], sem)
        return carry

    lax.fori_loop(0, GROUPS, issue, 0)

    def drain(g, carry):
        for i in range(16):
            e = base + g * 16 + i
            pltpu.make_async_copy(
                uemb_hbm.at[pl.ds(0, 1), :],
                stagu_hbm.at[pl.ds(e, 1), :], sem).wait()
            pltpu.make_async_copy(
                iemb_hbm.at[pl.ds(0, 1), :],
                stagi_hbm.at[pl.ds(e, 1), :], sem).wait()
            pltpu.make_async_copy(
                ubias_hbm.at[pl.ds(0, 1), :],
                stagub_hbm.at[pl.ds(e, 1), :], sem).wait()
            pltpu.make_async_copy(
                ibias_hbm.at[pl.ds(0, 1), :],
                stagib_hbm.at[pl.ds(e, 1), :], sem).wait()
        return carry

    lax.fori_loop(0, GROUPS, drain, 0)


def _compute_body(stagu_hbm, stagi_hbm, stagub_hbm, stagib_hbm,
                  gb_hbm, out_hbm, u_v, i_v, ub_v, ib_v, gb_v, out_v):
    wid = lax.axis_index("s") * NC + lax.axis_index("c")
    base = wid * B_PER_W

    pltpu.sync_copy(stagu_hbm.at[pl.ds(base, B_PER_W)], u_v)
    pltpu.sync_copy(stagi_hbm.at[pl.ds(base, B_PER_W)], i_v)
    pltpu.sync_copy(stagub_hbm.at[pl.ds(base, B_PER_W)], ub_v)
    pltpu.sync_copy(stagib_hbm.at[pl.ds(base, B_PER_W)], ib_v)
    pltpu.sync_copy(gb_hbm, gb_v)

    gb = gb_v[...]
    lane = lax.iota(jnp.int32, 16)
    zcol = jnp.zeros((16,), jnp.int32)

    def compute(g, carry):
        rows = lane + g * 16
        acc = (plsc.load_gather(ub_v, [rows, zcol])
               + plsc.load_gather(ib_v, [rows, zcol]))
        for c in range(EMB):
            ccol = jnp.full((16,), c, jnp.int32)
            u = plsc.load_gather(u_v, [rows, ccol])
            v = plsc.load_gather(i_v, [rows, ccol])
            acc = acc + u * v
        v = acc + gb
        out_v[pl.ds(g * 16, 16)] = 5.0 / (1.0 + jnp.exp(-2.0 * v))
        return carry

    lax.fori_loop(0, GROUPS, compute, 0)
    pltpu.sync_copy(out_v, out_hbm.at[pl.ds(base, B_PER_W)])


@jax.jit
def _mf(uid2d, iid2d, uemb, iemb, ubias, ibias, gb):
    mesh = plsc.VectorSubcoreMesh(core_axis_name="c", subcore_axis_name="s")
    gather = pl.kernel(
        _gather_body,
        out_type=(
            jax.ShapeDtypeStruct((BATCH, EMB), jnp.float32),
            jax.ShapeDtypeStruct((BATCH, EMB), jnp.float32),
            jax.ShapeDtypeStruct((BATCH, 1), jnp.float32),
            jax.ShapeDtypeStruct((BATCH, 1), jnp.float32),
        ),
        mesh=mesh,
        compiler_params=pltpu.CompilerParams(needs_layout_passes=False),
        scratch_types=[
            pltpu.VMEM((4, 128), jnp.int32),
            pltpu.VMEM((4, 128), jnp.int32),
            pltpu.SemaphoreType.DMA,
        ],
    )
    stagu, stagi, stagub, stagib = gather(uid2d, iid2d, uemb, iemb,
                                          ubias, ibias)

    compute = pl.kernel(
        _compute_body,
        out_type=jax.ShapeDtypeStruct((BATCH,), jnp.float32),
        mesh=mesh,
        compiler_params=pltpu.CompilerParams(needs_layout_passes=False,
                                             use_tc_tiling_on_sc=False),
        scratch_types=[
            pltpu.VMEM((B_PER_W, EMB), jnp.float32),
            pltpu.VMEM((B_PER_W, EMB), jnp.float32),
            pltpu.VMEM((B_PER_W, 1), jnp.float32),
            pltpu.VMEM((B_PER_W, 1), jnp.float32),
            pltpu.VMEM((16,), jnp.float32),
            pltpu.VMEM((B_PER_W,), jnp.float32),
        ],
    )
    return compute(stagu, stagi, stagub, stagib, gb)


def kernel(user_ids, item_ids, user_emb_table, item_emb_table,
           user_bias_table, item_bias_table, global_bias):
    gb16 = jnp.tile(global_bias.astype(jnp.float32), 16)
    uid2d = jnp.reshape(user_ids.astype(jnp.int32), (128, 128))
    iid2d = jnp.reshape(item_ids.astype(jnp.int32), (128, 128))
    return _mf(uid2d, iid2d, user_emb_table, item_emb_table,
               user_bias_table, item_bias_table, gb16)
